# baseline (speedup 1.0000x reference)
import functools

import jax
import jax.numpy as jnp
from jax import lax
from jax.experimental import pallas as pl
from jax.experimental.pallas import tpu as pltpu
from jax.experimental.pallas import tpu_sc as plsc

MAX_LEN = 8192
D_MODEL = 2048

_info = plsc.get_sparse_core_info()
_NC, _NS = _info.num_cores, _info.num_subcores
_NW = _NC * _NS
_ROWS_PER_W = MAX_LEN // _NW


@functools.partial(
    pl.kernel,
    mesh=plsc.VectorSubcoreMesh(core_axis_name="c", subcore_axis_name="s"),
    out_type=jax.ShapeDtypeStruct((1, MAX_LEN, D_MODEL), jnp.float32),
)
def _pos_embed_sc(table_hbm, out_hbm):
    wid = lax.axis_index("s") * _NC + lax.axis_index("c")
    base = wid * _ROWS_PER_W
    pltpu.sync_copy(
        table_hbm.at[pl.ds(base, _ROWS_PER_W), :],
        out_hbm.at[0, pl.ds(base, _ROWS_PER_W), :],
    )


def kernel(inputs, emb_table):
    del inputs
    return _pos_embed_sc(emb_table)


# speedup vs baseline: 31.7868x; 31.7868x over previous
import functools

import jax
import jax.numpy as jnp
from jax import lax
from jax.experimental import pallas as pl
from jax.experimental.pallas import tpu as pltpu
from jax.experimental.pallas import tpu_sc as plsc

MAX_LEN = 8192
D_MODEL = 2048

_info = plsc.get_sparse_core_info()
_NC, _NS = _info.num_cores, _info.num_subcores
_NW = _NC * _NS
_ROWS_PER_W = MAX_LEN // _NW


_C = 16
_NBUF = 3
_N_IT = _ROWS_PER_W // _C


@functools.partial(
    pl.kernel,
    mesh=plsc.VectorSubcoreMesh(core_axis_name="c", subcore_axis_name="s"),
    out_type=jax.ShapeDtypeStruct((1, MAX_LEN, D_MODEL), jnp.float32),
    scratch_types=[
        pltpu.VMEM((_NBUF, _C, D_MODEL), jnp.float32),
        pltpu.SemaphoreType.DMA((_NBUF,)),
        pltpu.SemaphoreType.DMA((_NBUF,)),
    ],
)
def _pos_embed_sc(table_hbm, out_hbm, buf, in_sem, out_sem):
    wid = lax.axis_index("s") * _NC + lax.axis_index("c")
    base = wid * _ROWS_PER_W

    def fetch(i, b):
        return pltpu.async_copy(
            table_hbm.at[pl.ds(base + i * _C, _C), :], buf.at[b], in_sem.at[b]
        )

    def drain(i, b):
        return pltpu.async_copy(
            buf.at[b], out_hbm.at[0, pl.ds(base + i * _C, _C), :], out_sem.at[b]
        )

    in_h = [None] * _N_IT
    out_h = [None] * _N_IT
    for j in range(_NBUF):
        in_h[j] = fetch(j, j)
    for i in range(_N_IT):
        b = i % _NBUF
        in_h[i].wait()
        out_h[i] = drain(i, b)
        nxt = i + _NBUF
        if nxt < _N_IT:
            out_h[i].wait()
            in_h[nxt] = fetch(nxt, b)
    for i in range(max(0, _N_IT - _NBUF), _N_IT):
        out_h[i].wait()


def kernel(inputs, emb_table):
    del inputs
    return _pos_embed_sc(emb_table)
